# baseline (device time: 41613 ns/iter reference)
import jax
import jax.numpy as jnp
from jax import lax
from jax.experimental import pallas as pl
from jax.experimental.pallas import tpu as pltpu

N_DEV = 4
B = 2
S_LOC = 256
D = 768
HQ = 4
DH = 64
HD = HQ * DH
ROWS = B * S_LOC


def _rope(t, pos, lane):
    j = lane % DH
    pair = (j // 2).astype(jnp.float32)
    inv = jnp.exp(pair * (-2.0 * jnp.log(10000.0) / DH))
    ang = pos * inv
    c = jnp.cos(ang)
    s = jnp.sin(ang)
    t_m = jnp.concatenate([t[:, 1:], t[:, :1]], axis=1)
    t_p = jnp.concatenate([t[:, -1:], t[:, :-1]], axis=1)
    t_r = jnp.where(lane % 2 == 0, -t_m, t_p)
    return t * c + t_r * s


def _body(x_ref, wq_ref, wk_ref, wv_ref, wo_ref, out_ref,
          kv_ref, send_sems, recv_sems):
    f32 = jnp.float32
    bf16 = jnp.bfloat16
    my_pos = lax.axis_index("i")
    left = lax.rem(my_pos + (N_DEV - 1), N_DEV)
    right = lax.rem(my_pos + 1, N_DEV)

    barrier = pltpu.get_barrier_semaphore()
    for nbr in (left, right):
        pl.semaphore_signal(barrier, inc=1, device_id=(nbr,),
                            device_id_type=pl.DeviceIdType.MESH)
    pl.semaphore_wait(barrier, 2)

    xb = x_ref[...].reshape(ROWS, D).astype(bf16)
    dn = (((1,), (0,)), ((), ()))
    q = lax.dot_general(xb, wq_ref[...].astype(bf16), dn,
                        preferred_element_type=f32)
    k = lax.dot_general(xb, wk_ref[...].astype(bf16), dn,
                        preferred_element_type=f32)
    v = lax.dot_general(xb, wv_ref[...].astype(bf16), dn,
                        preferred_element_type=f32)

    lane = lax.broadcasted_iota(jnp.int32, (ROWS, HD), 1)
    row = lax.broadcasted_iota(jnp.int32, (ROWS, HD), 0)
    pos = (lax.rem(row, S_LOC) + my_pos * S_LOC).astype(f32)
    q = _rope(q, pos, lane).astype(bf16)
    k = _rope(k, pos, lane).astype(bf16)
    v = v.astype(bf16)

    kv_ref[pl.ds(my_pos, 1)] = jnp.concatenate([k, v], axis=1)[None]

    for h in range(N_DEV - 1):
        so = lax.rem(my_pos + (N_DEV - h), N_DEV)
        rdma = pltpu.make_async_remote_copy(
            src_ref=kv_ref.at[so],
            dst_ref=kv_ref.at[so],
            send_sem=send_sems.at[h],
            recv_sem=recv_sems.at[h],
            device_id=(right,),
            device_id_type=pl.DeviceIdType.MESH,
        )
        rdma.start()
        rdma.wait()

    ctx_parts = []
    for b in range(B):
        kv_b = jnp.concatenate(
            [kv_ref[o, b * S_LOC:(b + 1) * S_LOC, :] for o in range(N_DEV)],
            axis=0)
        head_parts = []
        for hh in range(HQ):
            q_bh = q[b * S_LOC:(b + 1) * S_LOC, hh * DH:(hh + 1) * DH]
            k_bh = kv_b[:, hh * DH:(hh + 1) * DH]
            v_bh = kv_b[:, HD + hh * DH:HD + (hh + 1) * DH]
            s = lax.dot_general(q_bh, k_bh, (((1,), (1,)), ((), ())),
                                preferred_element_type=f32) * 0.125
            m = jnp.max(s, axis=1, keepdims=True)
            e = jnp.exp(s - m)
            denom = jnp.sum(e, axis=1, keepdims=True)
            ctx = lax.dot_general(e.astype(bf16), v_bh, dn,
                                  preferred_element_type=f32)
            head_parts.append((ctx / denom).astype(bf16))
        ctx_parts.append(jnp.concatenate(head_parts, axis=1))
    ctx_flat = jnp.concatenate(ctx_parts, axis=0)

    out = lax.dot_general(ctx_flat, wo_ref[...].astype(bf16), dn,
                          preferred_element_type=f32)
    out_ref[...] = out.reshape(B, S_LOC, D)


def kernel(x, Wq, Wk, Wv, Wo):
    return pl.pallas_call(
        _body,
        out_shape=jax.ShapeDtypeStruct((B, S_LOC, D), jnp.float32),
        in_specs=[pl.BlockSpec(memory_space=pltpu.VMEM)] * 5,
        out_specs=pl.BlockSpec(memory_space=pltpu.VMEM),
        scratch_shapes=[
            pltpu.VMEM((N_DEV, ROWS, 2 * HD), jnp.bfloat16),
            pltpu.SemaphoreType.DMA((N_DEV - 1,)),
            pltpu.SemaphoreType.DMA((N_DEV - 1,)),
        ],
        compiler_params=pltpu.CompilerParams(collective_id=0),
    )(x, Wq, Wk, Wv, Wo)


# device time: 27374 ns/iter; 1.5202x vs baseline; 1.5202x over previous
import jax
import jax.numpy as jnp
from jax import lax
from jax.experimental import pallas as pl
from jax.experimental.pallas import tpu as pltpu

N_DEV = 4
B = 2
S_LOC = 256
D = 768
HQ = 4
DH = 64
HD = HQ * DH
ROWS = B * S_LOC

_F32 = jnp.float32
_BF16 = jnp.bfloat16
_DN = (((1,), (0,)), ((), ()))
_DN_T = (((1,), (1,)), ((), ()))


def _rope(t, pos, lane):
    j = lane % DH
    pair = (j // 2).astype(_F32)
    inv = jnp.exp(pair * (-2.0 * jnp.log(10000.0) / DH))
    ang = pos * inv
    c = jnp.cos(ang)
    s = jnp.sin(ang)
    t_m = jnp.concatenate([t[:, 1:], t[:, :1]], axis=1)
    t_p = jnp.concatenate([t[:, -1:], t[:, :-1]], axis=1)
    t_r = jnp.where(lane % 2 == 0, -t_m, t_p)
    return t * c + t_r * s


def _attn_update(q, kvc, state):
    new = []
    for b in range(B):
        kb = kvc[b * S_LOC:(b + 1) * S_LOC, :]
        for hh in range(HQ):
            m, d, acc = state[b * HQ + hh]
            q_bh = q[b * S_LOC:(b + 1) * S_LOC, hh * DH:(hh + 1) * DH]
            k_bh = kb[:, hh * DH:(hh + 1) * DH]
            v_bh = kb[:, HD + hh * DH:HD + (hh + 1) * DH]
            s = lax.dot_general(q_bh, k_bh, _DN_T,
                                preferred_element_type=_F32) * 0.125
            mc = jnp.max(s, axis=1, keepdims=True)
            mn = jnp.maximum(m, mc)
            alpha = jnp.exp(m - mn)
            e = jnp.exp(s - mn)
            d = d * alpha + jnp.sum(e, axis=1, keepdims=True)
            acc = acc * alpha + lax.dot_general(
                e.astype(_BF16), v_bh, _DN, preferred_element_type=_F32)
            new.append((mn, d, acc))
    return new


def _body(x_ref, wq_ref, wk_ref, wv_ref, wo_ref, out_ref,
          kv_ref, send_sems, recv_sems):
    my_pos = lax.axis_index("i")
    left = lax.rem(my_pos + (N_DEV - 1), N_DEV)
    right = lax.rem(my_pos + 1, N_DEV)

    barrier = pltpu.get_barrier_semaphore()
    for nbr in (left, right):
        pl.semaphore_signal(barrier, inc=1, device_id=(nbr,),
                            device_id_type=pl.DeviceIdType.MESH)
    pl.semaphore_wait(barrier, 2)

    xb = x_ref[...].reshape(ROWS, D).astype(_BF16)
    k = lax.dot_general(xb, wk_ref[...].astype(_BF16), _DN,
                        preferred_element_type=_F32)
    v = lax.dot_general(xb, wv_ref[...].astype(_BF16), _DN,
                        preferred_element_type=_F32)

    lane = lax.broadcasted_iota(jnp.int32, (ROWS, HD), 1)
    row = lax.broadcasted_iota(jnp.int32, (ROWS, HD), 0)
    pos = (lax.rem(row, S_LOC) + my_pos * S_LOC).astype(_F32)
    kv0 = jnp.concatenate([_rope(k, pos, lane).astype(_BF16),
                           v.astype(_BF16)], axis=1)
    kv_ref[0] = kv0

    r1 = pltpu.make_async_remote_copy(
        src_ref=kv_ref.at[0], dst_ref=kv_ref.at[1],
        send_sem=send_sems.at[0], recv_sem=recv_sems.at[0],
        device_id=(right,), device_id_type=pl.DeviceIdType.MESH)
    l1 = pltpu.make_async_remote_copy(
        src_ref=kv_ref.at[0], dst_ref=kv_ref.at[2],
        send_sem=send_sems.at[1], recv_sem=recv_sems.at[1],
        device_id=(left,), device_id_type=pl.DeviceIdType.MESH)
    r1.start()
    l1.start()

    q = lax.dot_general(xb, wq_ref[...].astype(_BF16), _DN,
                        preferred_element_type=_F32)
    q = _rope(q, pos, lane).astype(_BF16)

    state = [(jnp.full((S_LOC, 1), -1e30, _F32),
              jnp.zeros((S_LOC, 1), _F32),
              jnp.zeros((S_LOC, DH), _F32)) for _ in range(B * HQ)]
    state = _attn_update(q, kv0, state)

    r1.wait_recv()
    r2 = pltpu.make_async_remote_copy(
        src_ref=kv_ref.at[1, pl.ds(0, S_LOC)],
        dst_ref=kv_ref.at[3, pl.ds(0, S_LOC)],
        send_sem=send_sems.at[2], recv_sem=recv_sems.at[2],
        device_id=(right,), device_id_type=pl.DeviceIdType.MESH)
    r2.start()
    l1.wait_recv()
    l2 = pltpu.make_async_remote_copy(
        src_ref=kv_ref.at[2, pl.ds(S_LOC, S_LOC)],
        dst_ref=kv_ref.at[3, pl.ds(S_LOC, S_LOC)],
        send_sem=send_sems.at[3], recv_sem=recv_sems.at[3],
        device_id=(left,), device_id_type=pl.DeviceIdType.MESH)
    l2.start()

    state = _attn_update(q, kv_ref[1], state)
    state = _attn_update(q, kv_ref[2], state)

    r2.wait_recv()
    l2.wait_recv()
    state = _attn_update(q, kv_ref[3], state)

    ctx_flat = jnp.concatenate(
        [jnp.concatenate(
            [(state[b * HQ + hh][2] / state[b * HQ + hh][1]).astype(_BF16)
             for hh in range(HQ)], axis=1)
         for b in range(B)], axis=0)
    out = lax.dot_general(ctx_flat, wo_ref[...].astype(_BF16), _DN,
                          preferred_element_type=_F32)
    out_ref[...] = out.reshape(B, S_LOC, D)

    r1.wait_send()
    l1.wait_send()
    r2.wait_send()
    l2.wait_send()


def kernel(x, Wq, Wk, Wv, Wo):
    return pl.pallas_call(
        _body,
        out_shape=jax.ShapeDtypeStruct((B, S_LOC, D), jnp.float32),
        in_specs=[pl.BlockSpec(memory_space=pltpu.VMEM)] * 5,
        out_specs=pl.BlockSpec(memory_space=pltpu.VMEM),
        scratch_shapes=[
            pltpu.VMEM((N_DEV, ROWS, 2 * HD), jnp.bfloat16),
            pltpu.SemaphoreType.DMA((4,)),
            pltpu.SemaphoreType.DMA((4,)),
        ],
        compiler_params=pltpu.CompilerParams(collective_id=0),
    )(x, Wq, Wk, Wv, Wo)
